# baseline (speedup 1.0000x reference)
import functools

import jax
import jax.numpy as jnp
from jax import lax
from jax.experimental import pallas as pl
from jax.experimental.pallas import tpu as pltpu
from jax.experimental.pallas import tpu_sc as plsc

N0, N1, N2 = 10000, 2000, 500
E0, E1, E2 = 320000, 64000, 16000
D = 128
DW = 16
EC = 128
NW = 32
PIPE = 8
N2P = 512


def _cdiv(a, b):
    return (a + b - 1) // b


_MESH = dict(core_axis_name="c", subcore_axis_name="s")


def _fire_drain_rows(src_of, dst_of, sem, nrc, rc, sid):
    nit = _cdiv(nrc, 16)

    def fire(i, c):
        ck = i * 16 + sid

        @pl.when(ck < nrc)
        def _():
            pltpu.async_copy(src_of(ck), dst_of(ck), sem)

        return c

    lax.fori_loop(0, nit, fire, 0)

    def drain(i, c):
        ck = i * 16 + sid

        @pl.when(ck < nrc)
        def _():
            pltpu.make_async_copy(src_of(ck), dst_of(ck), sem).wait()

        return c

    lax.fori_loop(0, nit, drain, 0)


def _sc_agg(N, E, rc, ec, pipe):
    nec = E // ec
    ne_it = _cdiv(nec, NW)
    ng = _cdiv(ne_it, pipe)
    nrc = N // rc

    @functools.partial(
        pl.kernel,
        mesh=plsc.VectorSubcoreMesh(**_MESH),
        out_type=jax.ShapeDtypeStruct((2 * N, D), jnp.float32),
        scratch_types=(
            [pltpu.VMEM((ec,), jnp.int32) for _ in range(2 * pipe)]
            + [pltpu.VMEM((ec, D), jnp.float32) for _ in range(pipe)]
            + [pltpu.VMEM_SHARED((N, D), jnp.float32)]
            + [pltpu.SemaphoreType.DMA for _ in range(3 * pipe + 1)]
        ),
    )
    def k(m_hbm, src_hbm, dst_hbm, zer_hbm, out_hbm, *sc):
        sidx = sc[0:pipe]
        didx = sc[pipe:2 * pipe]
        rows = sc[2 * pipe:3 * pipe]
        acc = sc[3 * pipe]
        semi = sc[3 * pipe + 1:4 * pipe + 1]
        semg = sc[4 * pipe + 1:5 * pipe + 1]
        semw = sc[5 * pipe + 1:6 * pipe + 1]
        semz = sc[6 * pipe + 1]
        cid = lax.axis_index("c")
        sid = lax.axis_index("s")
        wid = sid * 2 + cid
        pltpu.sync_copy(zer_hbm, rows[0])
        _fire_drain_rows(lambda ck: rows[0].at[pl.ds(0, rc)],
                         lambda ck: acc.at[pl.ds(ck * rc, rc)],
                         semz, nrc, rc, sid)
        plsc.subcore_barrier()

        def ebody(g, c):
            def chunk(b):
                return (g * pipe + b) * NW + wid

            for b in range(pipe):
                ck = chunk(b)

                @pl.when(ck < nec)
                def _(ck=ck, b=b):
                    base = ck * ec
                    pltpu.async_copy(src_hbm.at[pl.ds(base, ec)], sidx[b], semi[b])
                    pltpu.async_copy(dst_hbm.at[pl.ds(base, ec)], didx[b], semi[b])

            for b in range(pipe):
                ck = chunk(b)

                @pl.when(ck < nec)
                def _(ck=ck, b=b):
                    base = ck * ec
                    pltpu.make_async_copy(src_hbm.at[pl.ds(base, ec)], sidx[b], semi[b]).wait()
                    pltpu.make_async_copy(dst_hbm.at[pl.ds(base, ec)], didx[b], semi[b]).wait()
                    pltpu.async_copy(m_hbm.at[sidx[b]], rows[b], semg[b])

            for b in range(pipe):
                ck = chunk(b)

                @pl.when(ck < nec)
                def _(ck=ck, b=b):
                    pltpu.make_async_copy(m_hbm.at[sidx[b]], rows[b], semg[b]).wait()
                    pltpu.async_copy(rows[b], acc.at[didx[b]], semw[b], add=True)

            for b in range(pipe):
                ck = chunk(b)

                @pl.when(ck < nec)
                def _(ck=ck, b=b):
                    pltpu.make_async_copy(rows[b], acc.at[didx[b]], semw[b]).wait()

            return c

        lax.fori_loop(0, ng, ebody, 0)
        plsc.subcore_barrier()
        _fire_drain_rows(lambda ck: acc.at[pl.ds(ck * rc, rc)],
                         lambda ck: out_hbm.at[pl.ds(cid * N + ck * rc, rc)],
                         semz, nrc, rc, sid)

    return k


_DEG_STREAMS = [(E0, N0, 80), (E0, N0, 80), (E1, N1, 80),
                (E1, N1, 80), (E2, N2P, 64), (E2, N2P, 64)]


@functools.partial(
    pl.kernel,
    mesh=plsc.VectorSubcoreMesh(**_MESH),
    out_type=[jax.ShapeDtypeStruct((2 * n, DW), jnp.float32)
              for (_, n, _) in _DEG_STREAMS],
    scratch_types=(
        [pltpu.VMEM((EC,), jnp.int32) for _ in range(PIPE)]
        + [pltpu.VMEM((EC, DW), jnp.float32), pltpu.VMEM((EC, DW), jnp.float32)]
        + [pltpu.VMEM_SHARED((n, DW), jnp.float32) for (_, n, _) in _DEG_STREAMS]
        + [pltpu.SemaphoreType.DMA for _ in range(2 * PIPE + 1)]
    ),
    compiler_params=pltpu.CompilerParams(use_tc_tiling_on_sc=False),
)
def _sc_deg(i0s, i0d, i1s, i1d, i2s, i2d, ones_hbm, zer_hbm,
            o0s, o0d, o1s, o1d, o2s, o2d, *sc):
    idxb = sc[0:PIPE]
    onesb = sc[PIPE]
    zb = sc[PIPE + 1]
    accs = sc[PIPE + 2:PIPE + 8]
    semi = sc[PIPE + 8:2 * PIPE + 8]
    semw = sc[2 * PIPE + 8:3 * PIPE + 8]
    semz = sc[3 * PIPE + 8]
    cid = lax.axis_index("c")
    sid = lax.axis_index("s")
    wid = sid * 2 + cid
    pltpu.sync_copy(ones_hbm, onesb)
    pltpu.sync_copy(zer_hbm, zb)
    idxs = [i0s, i0d, i1s, i1d, i2s, i2d]
    outs = [o0s, o0d, o1s, o1d, o2s, o2d]
    for (e, n, rc), acc in zip(_DEG_STREAMS, accs):
        _fire_drain_rows(lambda ck, rc=rc: zb.at[pl.ds(0, rc)],
                         lambda ck, acc=acc, rc=rc: acc.at[pl.ds(ck * rc, rc)],
                         semz, n // rc, rc, sid)
    plsc.subcore_barrier()
    for (e, n, rc), idx, acc in zip(_DEG_STREAMS, idxs, accs):
        nec = e // EC
        ng = _cdiv(_cdiv(nec, NW), PIPE)

        def ebody(g, c, idx=idx, acc=acc, nec=nec):
            def chunk(b):
                return (g * PIPE + b) * NW + wid

            for b in range(PIPE):
                ck = chunk(b)

                @pl.when(ck < nec)
                def _(ck=ck, b=b):
                    pltpu.async_copy(idx.at[pl.ds(ck * EC, EC)], idxb[b], semi[b])

            for b in range(PIPE):
                ck = chunk(b)

                @pl.when(ck < nec)
                def _(ck=ck, b=b):
                    pltpu.make_async_copy(idx.at[pl.ds(ck * EC, EC)], idxb[b], semi[b]).wait()
                    pltpu.async_copy(onesb, acc.at[idxb[b]], semw[b], add=True)

            for b in range(PIPE):
                ck = chunk(b)

                @pl.when(ck < nec)
                def _(ck=ck, b=b):
                    pltpu.make_async_copy(onesb, acc.at[idxb[b]], semw[b]).wait()

            return c

        lax.fori_loop(0, ng, ebody, 0)
    plsc.subcore_barrier()
    for (e, n, rc), acc, out in zip(_DEG_STREAMS, accs, outs):
        _fire_drain_rows(lambda ck, acc=acc, rc=rc: acc.at[pl.ds(ck * rc, rc)],
                         lambda ck, out=out, rc=rc, n=n: out.at[pl.ds(cid * n + ck * rc, rc)],
                         semz, n // rc, rc, sid)


def _norm(dref):
    d = dref[0, :, 0:1] + dref[1, :, 0:1]
    return jnp.where(d > 0, lax.rsqrt(jnp.maximum(d, 1.0)), 0.0)


def _tc_scale_m0(features, dS0):
    R = 400

    def body(fref, dref, oref):
        oref[...] = fref[...] * _norm(dref)

    return pl.pallas_call(
        body,
        grid=(N0 // R,),
        in_specs=[pl.BlockSpec((R, D), lambda i: (i, 0)),
                  pl.BlockSpec((2, R, DW), lambda i: (0, i, 0))],
        out_specs=pl.BlockSpec((R, D), lambda i: (i, 0)),
        out_shape=jax.ShapeDtypeStruct((N0, D), jnp.float32),
    )(features, dS0)


def _tc_gcn(aggp, dD, W, b, relu, N, Dout, R):
    def body(aref, dref, wref, bref, oref):
        a = (aref[0] + aref[1]) * _norm(dref)
        o = jnp.dot(a, wref[...], preferred_element_type=jnp.float32) + bref[...]
        if relu:
            o = jnp.maximum(o, 0.0)
        oref[...] = o

    return pl.pallas_call(
        body,
        grid=(N // R,),
        in_specs=[pl.BlockSpec((2, R, D), lambda i: (0, i, 0)),
                  pl.BlockSpec((2, R, DW), lambda i: (0, i, 0)),
                  pl.BlockSpec((D, Dout), lambda i: (0, 0)),
                  pl.BlockSpec((1, Dout), lambda i: (0, 0))],
        out_specs=pl.BlockSpec((R, Dout), lambda i: (i, 0)),
        out_shape=jax.ShapeDtypeStruct((N, Dout), jnp.float32),
    )(aggp, dD, W, b)


def _tc_gcn_proj(aggp, dD, W, b, P, dS, Nout, K, Kb):
    nk = K // Kb

    def body(aref, dref, wref, bref, pref, sref, oref):
        kk = pl.program_id(0)
        a = (aref[0] + aref[1]) * _norm(dref)
        h = jnp.maximum(
            jnp.dot(a, wref[...], preferred_element_type=jnp.float32)
            + bref[...], 0.0)
        t = lax.dot_general(pref[...], h, (((0,), (0,)), ((), ())),
                            preferred_element_type=jnp.float32)

        @pl.when(kk == 0)
        def _():
            oref[...] = t

        @pl.when(kk > 0)
        def _():
            oref[...] += t

        @pl.when(kk == nk - 1)
        def _():
            oref[...] *= _norm(sref)

    return pl.pallas_call(
        body,
        grid=(nk,),
        in_specs=[pl.BlockSpec((2, Kb, D), lambda k: (0, k, 0)),
                  pl.BlockSpec((2, Kb, DW), lambda k: (0, k, 0)),
                  pl.BlockSpec((D, D), lambda k: (0, 0)),
                  pl.BlockSpec((1, D), lambda k: (0, 0)),
                  pl.BlockSpec((Kb, Nout), lambda k: (k, 0)),
                  pl.BlockSpec((2, Nout, DW), lambda k: (0, 0, 0))],
        out_specs=pl.BlockSpec((Nout, D), lambda k: (0, 0)),
        out_shape=jax.ShapeDtypeStruct((Nout, D), jnp.float32),
    )(aggp, dD, W, b, P, dS)


def _tc_gcn_proj_emb(aggp, dD, W, b, P2p, dS2p):
    def body(aref, dref, wref, bref, pref, sref, eref, mref):
        a = (aref[0] + aref[1]) * _norm(dref)
        h = jnp.maximum(
            jnp.dot(a, wref[...], preferred_element_type=jnp.float32)
            + bref[...], 0.0)
        e = lax.dot_general(pref[...], h, (((0,), (0,)), ((), ())),
                            preferred_element_type=jnp.float32)
        eref[...] = e
        mref[...] = e * _norm(sref)

    return pl.pallas_call(
        body,
        grid=(1,),
        in_specs=[pl.BlockSpec((2, N1, D), lambda i: (0, 0, 0)),
                  pl.BlockSpec((2, N1, DW), lambda i: (0, 0, 0)),
                  pl.BlockSpec((D, D), lambda i: (0, 0)),
                  pl.BlockSpec((1, D), lambda i: (0, 0)),
                  pl.BlockSpec((N1, N2P), lambda i: (0, 0)),
                  pl.BlockSpec((2, N2P, DW), lambda i: (0, 0, 0))],
        out_specs=[pl.BlockSpec((N2P, D), lambda i: (0, 0)),
                   pl.BlockSpec((N2P, D), lambda i: (0, 0))],
        out_shape=[jax.ShapeDtypeStruct((N2P, D), jnp.float32),
                   jax.ShapeDtypeStruct((N2P, D), jnp.float32)],
    )(aggp, dD, W, b, P2p, dS2p)


def kernel(features, edge_index0, edge_index1, edge_index2, P1, P2,
           W0, b0, W1, b1, W2, b2):
    f32 = jnp.float32
    src0 = edge_index0[0].astype(jnp.int32)
    dst0 = edge_index0[1].astype(jnp.int32)
    src1 = edge_index1[0].astype(jnp.int32)
    dst1 = edge_index1[1].astype(jnp.int32)
    src2 = edge_index2[0].astype(jnp.int32)
    dst2 = edge_index2[1].astype(jnp.int32)
    z128 = jnp.zeros((EC, D), f32)
    z16 = jnp.zeros((EC, DW), f32)
    o16 = jnp.ones((EC, DW), f32)

    degs = _sc_deg(src0, dst0, src1, dst1, src2, dst2, o16, z16)
    dS0, dD0, dS1, dD1, dS2p, dD2p = [x.reshape(2, -1, DW) for x in degs]

    z64 = jnp.zeros((64, D), f32)
    m0 = _tc_scale_m0(features, dS0)
    agg0 = _sc_agg(N0, E0, 40, 128, 3)(m0, src0, dst0, z128).reshape(2, N0, D)
    m1 = _tc_gcn_proj(agg0, dD0, W0, b0.reshape(1, -1), P1, dS1, N1, N0, 1000)
    agg1 = _sc_agg(N1, E1, 80, EC, 6)(m1, src1, dst1, z128).reshape(2, N1, D)
    P2p = jnp.pad(P2, ((0, 0), (0, N2P - N2)))
    emb_p, m2_p = _tc_gcn_proj_emb(agg1, dD1, W1, b1.reshape(1, -1), P2p, dS2p)
    agg2p = _sc_agg(N2P, E2, 64, EC, 6)(m2_p, src2, dst2, z128).reshape(2, N2P, D)
    outp = _tc_gcn(agg2p, dD2p, W2, b2.reshape(1, -1), False, N2P, 64, N2P)
    return (outp[:N2], emb_p[:N2])


# speedup vs baseline: 1.0291x; 1.0291x over previous
import functools

import jax
import jax.numpy as jnp
from jax import lax
from jax.experimental import pallas as pl
from jax.experimental.pallas import tpu as pltpu
from jax.experimental.pallas import tpu_sc as plsc

N0, N1, N2 = 10000, 2000, 500
E0, E1, E2 = 320000, 64000, 16000
D = 128
DW = 16
EC = 128
NW = 32
PIPE = 8
N2P = 512


def _cdiv(a, b):
    return (a + b - 1) // b


_MESH = dict(core_axis_name="c", subcore_axis_name="s")


def _fire_drain_rows(src_of, dst_of, sem, nrc, rc, sid):
    nit = _cdiv(nrc, 16)

    def fire(i, c):
        ck = i * 16 + sid

        @pl.when(ck < nrc)
        def _():
            pltpu.async_copy(src_of(ck), dst_of(ck), sem)

        return c

    lax.fori_loop(0, nit, fire, 0)

    def drain(i, c):
        ck = i * 16 + sid

        @pl.when(ck < nrc)
        def _():
            pltpu.make_async_copy(src_of(ck), dst_of(ck), sem).wait()

        return c

    lax.fori_loop(0, nit, drain, 0)


def _sc_agg(N, E, rc, ec, pipe, d=D):
    nec = E // ec
    ne_it = _cdiv(nec, NW)
    ng = _cdiv(ne_it, pipe)
    nrc = N // rc

    @functools.partial(
        pl.kernel,
        mesh=plsc.VectorSubcoreMesh(**_MESH),
        out_type=jax.ShapeDtypeStruct((2 * N, d), jnp.float32),
        scratch_types=(
            [pltpu.VMEM((ec,), jnp.int32) for _ in range(2 * pipe)]
            + [pltpu.VMEM((ec, d), jnp.float32) for _ in range(pipe)]
            + [pltpu.VMEM_SHARED((N, d), jnp.float32)]
            + [pltpu.SemaphoreType.DMA for _ in range(3 * pipe + 1)]
        ),
        compiler_params=pltpu.CompilerParams(use_tc_tiling_on_sc=(d == D)),
    )
    def k(m_hbm, src_hbm, dst_hbm, zer_hbm, out_hbm, *sc):
        sidx = sc[0:pipe]
        didx = sc[pipe:2 * pipe]
        rows = sc[2 * pipe:3 * pipe]
        acc = sc[3 * pipe]
        semi = sc[3 * pipe + 1:4 * pipe + 1]
        semg = sc[4 * pipe + 1:5 * pipe + 1]
        semw = sc[5 * pipe + 1:6 * pipe + 1]
        semz = sc[6 * pipe + 1]
        cid = lax.axis_index("c")
        sid = lax.axis_index("s")
        wid = sid * 2 + cid
        pltpu.sync_copy(zer_hbm, rows[0])
        _fire_drain_rows(lambda ck: rows[0].at[pl.ds(0, rc)],
                         lambda ck: acc.at[pl.ds(ck * rc, rc)],
                         semz, nrc, rc, sid)
        plsc.subcore_barrier()

        def ebody(g, c):
            def chunk(b):
                return (g * pipe + b) * NW + wid

            for b in range(pipe):
                ck = chunk(b)

                @pl.when(ck < nec)
                def _(ck=ck, b=b):
                    base = ck * ec
                    pltpu.async_copy(src_hbm.at[pl.ds(base, ec)], sidx[b], semi[b])
                    pltpu.async_copy(dst_hbm.at[pl.ds(base, ec)], didx[b], semi[b])

            for b in range(pipe):
                ck = chunk(b)

                @pl.when(ck < nec)
                def _(ck=ck, b=b):
                    base = ck * ec
                    pltpu.make_async_copy(src_hbm.at[pl.ds(base, ec)], sidx[b], semi[b]).wait()
                    pltpu.make_async_copy(dst_hbm.at[pl.ds(base, ec)], didx[b], semi[b]).wait()
                    pltpu.async_copy(m_hbm.at[sidx[b]], rows[b], semg[b])

            for b in range(pipe):
                ck = chunk(b)

                @pl.when(ck < nec)
                def _(ck=ck, b=b):
                    pltpu.make_async_copy(m_hbm.at[sidx[b]], rows[b], semg[b]).wait()
                    pltpu.async_copy(rows[b], acc.at[didx[b]], semw[b], add=True)

            for b in range(pipe):
                ck = chunk(b)

                @pl.when(ck < nec)
                def _(ck=ck, b=b):
                    pltpu.make_async_copy(rows[b], acc.at[didx[b]], semw[b]).wait()

            return c

        lax.fori_loop(0, ng, ebody, 0)
        plsc.subcore_barrier()
        _fire_drain_rows(lambda ck: acc.at[pl.ds(ck * rc, rc)],
                         lambda ck: out_hbm.at[pl.ds(cid * N + ck * rc, rc)],
                         semz, nrc, rc, sid)

    return k


_DEG_STREAMS = [(E0, N0, 80), (E0, N0, 80), (E1, N1, 80),
                (E1, N1, 80), (E2, N2P, 64), (E2, N2P, 64)]


@functools.partial(
    pl.kernel,
    mesh=plsc.VectorSubcoreMesh(**_MESH),
    out_type=[jax.ShapeDtypeStruct((2 * n, DW), jnp.float32)
              for (_, n, _) in _DEG_STREAMS],
    scratch_types=(
        [pltpu.VMEM((EC,), jnp.int32) for _ in range(PIPE)]
        + [pltpu.VMEM((EC, DW), jnp.float32), pltpu.VMEM((EC, DW), jnp.float32)]
        + [pltpu.VMEM_SHARED((n, DW), jnp.float32) for (_, n, _) in _DEG_STREAMS]
        + [pltpu.SemaphoreType.DMA for _ in range(2 * PIPE + 1)]
    ),
    compiler_params=pltpu.CompilerParams(use_tc_tiling_on_sc=False),
)
def _sc_deg(i0s, i0d, i1s, i1d, i2s, i2d, ones_hbm, zer_hbm,
            o0s, o0d, o1s, o1d, o2s, o2d, *sc):
    idxb = sc[0:PIPE]
    onesb = sc[PIPE]
    zb = sc[PIPE + 1]
    accs = sc[PIPE + 2:PIPE + 8]
    semi = sc[PIPE + 8:2 * PIPE + 8]
    semw = sc[2 * PIPE + 8:3 * PIPE + 8]
    semz = sc[3 * PIPE + 8]
    cid = lax.axis_index("c")
    sid = lax.axis_index("s")
    wid = sid * 2 + cid
    pltpu.sync_copy(ones_hbm, onesb)
    pltpu.sync_copy(zer_hbm, zb)
    idxs = [i0s, i0d, i1s, i1d, i2s, i2d]
    outs = [o0s, o0d, o1s, o1d, o2s, o2d]
    for (e, n, rc), acc in zip(_DEG_STREAMS, accs):
        _fire_drain_rows(lambda ck, rc=rc: zb.at[pl.ds(0, rc)],
                         lambda ck, acc=acc, rc=rc: acc.at[pl.ds(ck * rc, rc)],
                         semz, n // rc, rc, sid)
    plsc.subcore_barrier()
    for (e, n, rc), idx, acc in zip(_DEG_STREAMS, idxs, accs):
        nec = e // EC
        ng = _cdiv(_cdiv(nec, NW), PIPE)

        def ebody(g, c, idx=idx, acc=acc, nec=nec):
            def chunk(b):
                return (g * PIPE + b) * NW + wid

            for b in range(PIPE):
                ck = chunk(b)

                @pl.when(ck < nec)
                def _(ck=ck, b=b):
                    pltpu.async_copy(idx.at[pl.ds(ck * EC, EC)], idxb[b], semi[b])

            for b in range(PIPE):
                ck = chunk(b)

                @pl.when(ck < nec)
                def _(ck=ck, b=b):
                    pltpu.make_async_copy(idx.at[pl.ds(ck * EC, EC)], idxb[b], semi[b]).wait()
                    pltpu.async_copy(onesb, acc.at[idxb[b]], semw[b], add=True)

            for b in range(PIPE):
                ck = chunk(b)

                @pl.when(ck < nec)
                def _(ck=ck, b=b):
                    pltpu.make_async_copy(onesb, acc.at[idxb[b]], semw[b]).wait()

            return c

        lax.fori_loop(0, ng, ebody, 0)
    plsc.subcore_barrier()
    for (e, n, rc), acc, out in zip(_DEG_STREAMS, accs, outs):
        _fire_drain_rows(lambda ck, acc=acc, rc=rc: acc.at[pl.ds(ck * rc, rc)],
                         lambda ck, out=out, rc=rc, n=n: out.at[pl.ds(cid * n + ck * rc, rc)],
                         semz, n // rc, rc, sid)


def _norm(dref):
    d = dref[0, :, 0:1] + dref[1, :, 0:1]
    return jnp.where(d > 0, lax.rsqrt(jnp.maximum(d, 1.0)), 0.0)


def _tc_scale_m0(features, dS0):
    R = 400

    def body(fref, dref, oref):
        oref[...] = fref[...] * _norm(dref)

    return pl.pallas_call(
        body,
        grid=(N0 // R,),
        in_specs=[pl.BlockSpec((R, D), lambda i: (i, 0)),
                  pl.BlockSpec((2, R, DW), lambda i: (0, i, 0))],
        out_specs=pl.BlockSpec((R, D), lambda i: (i, 0)),
        out_shape=jax.ShapeDtypeStruct((N0, D), jnp.float32),
    )(features, dS0)


def _tc_gcn(aggp, dD, W, b, relu, N, Dout, R):
    def body(aref, dref, wref, bref, oref):
        a = (aref[0] + aref[1]) * _norm(dref)
        o = jnp.dot(a, wref[...], preferred_element_type=jnp.float32) + bref[...]
        if relu:
            o = jnp.maximum(o, 0.0)
        oref[...] = o

    return pl.pallas_call(
        body,
        grid=(N // R,),
        in_specs=[pl.BlockSpec((2, R, D), lambda i: (0, i, 0)),
                  pl.BlockSpec((2, R, DW), lambda i: (0, i, 0)),
                  pl.BlockSpec((D, Dout), lambda i: (0, 0)),
                  pl.BlockSpec((1, Dout), lambda i: (0, 0))],
        out_specs=pl.BlockSpec((R, Dout), lambda i: (i, 0)),
        out_shape=jax.ShapeDtypeStruct((N, Dout), jnp.float32),
    )(aggp, dD, W, b)


def _tc_gcn_proj(aggp, dD, W, b, P, dS, Nout, K, Kb):
    nk = K // Kb

    def body(aref, dref, wref, bref, pref, sref, oref):
        kk = pl.program_id(0)
        a = (aref[0] + aref[1]) * _norm(dref)
        h = jnp.maximum(
            jnp.dot(a, wref[...], preferred_element_type=jnp.float32)
            + bref[...], 0.0)
        t = lax.dot_general(pref[...], h, (((0,), (0,)), ((), ())),
                            preferred_element_type=jnp.float32)

        @pl.when(kk == 0)
        def _():
            oref[...] = t

        @pl.when(kk > 0)
        def _():
            oref[...] += t

        @pl.when(kk == nk - 1)
        def _():
            oref[...] *= _norm(sref)

    return pl.pallas_call(
        body,
        grid=(nk,),
        in_specs=[pl.BlockSpec((2, Kb, D), lambda k: (0, k, 0)),
                  pl.BlockSpec((2, Kb, DW), lambda k: (0, k, 0)),
                  pl.BlockSpec((D, D), lambda k: (0, 0)),
                  pl.BlockSpec((1, D), lambda k: (0, 0)),
                  pl.BlockSpec((Kb, Nout), lambda k: (k, 0)),
                  pl.BlockSpec((2, Nout, DW), lambda k: (0, 0, 0))],
        out_specs=pl.BlockSpec((Nout, D), lambda k: (0, 0)),
        out_shape=jax.ShapeDtypeStruct((Nout, D), jnp.float32),
    )(aggp, dD, W, b, P, dS)


def _tc_gcn_proj_emb(aggp, dD, W, b, P2p, dS2p, W2):
    def body(aref, dref, wref, bref, pref, sref, w2ref, eref, mref):
        a = (aref[0] + aref[1]) * _norm(dref)
        h = jnp.maximum(
            jnp.dot(a, wref[...], preferred_element_type=jnp.float32)
            + bref[...], 0.0)
        e = lax.dot_general(pref[...], h, (((0,), (0,)), ((), ())),
                            preferred_element_type=jnp.float32)
        eref[...] = e
        mref[...] = jnp.dot(e * _norm(sref), w2ref[...],
                            preferred_element_type=jnp.float32)

    return pl.pallas_call(
        body,
        grid=(1,),
        in_specs=[pl.BlockSpec((2, N1, D), lambda i: (0, 0, 0)),
                  pl.BlockSpec((2, N1, DW), lambda i: (0, 0, 0)),
                  pl.BlockSpec((D, D), lambda i: (0, 0)),
                  pl.BlockSpec((1, D), lambda i: (0, 0)),
                  pl.BlockSpec((N1, N2P), lambda i: (0, 0)),
                  pl.BlockSpec((2, N2P, DW), lambda i: (0, 0, 0)),
                  pl.BlockSpec((D, 64), lambda i: (0, 0))],
        out_specs=[pl.BlockSpec((N2P, D), lambda i: (0, 0)),
                   pl.BlockSpec((N2P, 64), lambda i: (0, 0))],
        out_shape=[jax.ShapeDtypeStruct((N2P, D), jnp.float32),
                   jax.ShapeDtypeStruct((N2P, 64), jnp.float32)],
    )(aggp, dD, W, b, P2p, dS2p, W2)


def _tc_final(aggp, dD, b):
    def body(aref, dref, bref, oref):
        oref[...] = (aref[0] + aref[1]) * _norm(dref) + bref[...]

    return pl.pallas_call(
        body,
        grid=(1,),
        in_specs=[pl.BlockSpec((2, N2P, 64), lambda i: (0, 0, 0)),
                  pl.BlockSpec((2, N2P, DW), lambda i: (0, 0, 0)),
                  pl.BlockSpec((1, 64), lambda i: (0, 0))],
        out_specs=pl.BlockSpec((N2P, 64), lambda i: (0, 0)),
        out_shape=jax.ShapeDtypeStruct((N2P, 64), jnp.float32),
    )(aggp, dD, b)


def kernel(features, edge_index0, edge_index1, edge_index2, P1, P2,
           W0, b0, W1, b1, W2, b2):
    f32 = jnp.float32
    src0 = edge_index0[0].astype(jnp.int32)
    dst0 = edge_index0[1].astype(jnp.int32)
    src1 = edge_index1[0].astype(jnp.int32)
    dst1 = edge_index1[1].astype(jnp.int32)
    src2 = edge_index2[0].astype(jnp.int32)
    dst2 = edge_index2[1].astype(jnp.int32)
    z128 = jnp.zeros((EC, D), f32)
    z16 = jnp.zeros((EC, DW), f32)
    o16 = jnp.ones((EC, DW), f32)

    degs = _sc_deg(src0, dst0, src1, dst1, src2, dst2, o16, z16)
    dS0, dD0, dS1, dD1, dS2p, dD2p = [x.reshape(2, -1, DW) for x in degs]

    z64 = jnp.zeros((64, D), f32)
    m0 = _tc_scale_m0(features, dS0)
    agg0 = _sc_agg(N0, E0, 40, 64, 6)(m0, src0, dst0, z64).reshape(2, N0, D)
    m1 = _tc_gcn_proj(agg0, dD0, W0, b0.reshape(1, -1), P1, dS1, N1, N0, 1000)
    agg1 = _sc_agg(N1, E1, 80, EC, 6)(m1, src1, dst1, z128).reshape(2, N1, D)
    P2p = jnp.pad(P2, ((0, 0), (0, N2P - N2)))
    emb_p, m2w = _tc_gcn_proj_emb(agg1, dD1, W1, b1.reshape(1, -1), P2p,
                                  dS2p, W2)
    z64w = jnp.zeros((EC, 64), f32)
    agg2p = _sc_agg(N2P, E2, 64, EC, 6, 64)(m2w, src2, dst2,
                                            z64w).reshape(2, N2P, 64)
    outp = _tc_final(agg2p, dD2p, b2.reshape(1, -1))
    return (outp[:N2], emb_p[:N2])


# speedup vs baseline: 1.0447x; 1.0151x over previous
import functools

import jax
import jax.numpy as jnp
from jax import lax
from jax.experimental import pallas as pl
from jax.experimental.pallas import tpu as pltpu
from jax.experimental.pallas import tpu_sc as plsc

N0, N1, N2 = 10000, 2000, 500
E0, E1, E2 = 320000, 64000, 16000
D = 128
DW = 16
EC = 128
NW = 32
PIPE = 8
N2P = 512


def _cdiv(a, b):
    return (a + b - 1) // b


_MESH = dict(core_axis_name="c", subcore_axis_name="s")


def _fire_drain_rows(src_of, dst_of, sem, nrc, rc, sid):
    nit = _cdiv(nrc, 16)

    def fire(i, c):
        ck = i * 16 + sid

        @pl.when(ck < nrc)
        def _():
            pltpu.async_copy(src_of(ck), dst_of(ck), sem)

        return c

    lax.fori_loop(0, nit, fire, 0)

    def drain(i, c):
        ck = i * 16 + sid

        @pl.when(ck < nrc)
        def _():
            pltpu.make_async_copy(src_of(ck), dst_of(ck), sem).wait()

        return c

    lax.fori_loop(0, nit, drain, 0)


def _sc_agg(N, E, rc, ec, pipe, d=D):
    nec = E // ec
    ne_it = _cdiv(nec, NW)
    ng = _cdiv(ne_it, pipe)
    nrc = N // rc

    @functools.partial(
        pl.kernel,
        mesh=plsc.VectorSubcoreMesh(**_MESH),
        out_type=jax.ShapeDtypeStruct((2 * N, d), jnp.float32),
        scratch_types=(
            [pltpu.VMEM((ec,), jnp.int32) for _ in range(2 * pipe)]
            + [pltpu.VMEM((ec, d), jnp.float32) for _ in range(pipe)]
            + [pltpu.VMEM_SHARED((N, d), jnp.float32)]
            + [pltpu.SemaphoreType.DMA for _ in range(3 * pipe + 1)]
        ),
        compiler_params=pltpu.CompilerParams(use_tc_tiling_on_sc=(d == D)),
    )
    def k(m_hbm, src_hbm, dst_hbm, zer_hbm, out_hbm, *sc):
        sidx = sc[0:pipe]
        didx = sc[pipe:2 * pipe]
        rows = sc[2 * pipe:3 * pipe]
        acc = sc[3 * pipe]
        semi = sc[3 * pipe + 1:4 * pipe + 1]
        semg = sc[4 * pipe + 1:5 * pipe + 1]
        semw = sc[5 * pipe + 1:6 * pipe + 1]
        semz = sc[6 * pipe + 1]
        cid = lax.axis_index("c")
        sid = lax.axis_index("s")
        wid = sid * 2 + cid
        pltpu.sync_copy(zer_hbm, rows[0])
        _fire_drain_rows(lambda ck: rows[0].at[pl.ds(0, rc)],
                         lambda ck: acc.at[pl.ds(ck * rc, rc)],
                         semz, nrc, rc, sid)
        plsc.subcore_barrier()

        def ebody(g, c):
            def chunk(b):
                return (g * pipe + b) * NW + wid

            for b in range(pipe):
                ck = chunk(b)

                @pl.when(ck < nec)
                def _(ck=ck, b=b):
                    base = ck * ec
                    pltpu.async_copy(src_hbm.at[pl.ds(base, ec)], sidx[b], semi[b])
                    pltpu.async_copy(dst_hbm.at[pl.ds(base, ec)], didx[b], semi[b])

            for b in range(pipe):
                ck = chunk(b)

                @pl.when(ck < nec)
                def _(ck=ck, b=b):
                    base = ck * ec
                    pltpu.make_async_copy(src_hbm.at[pl.ds(base, ec)], sidx[b], semi[b]).wait()
                    pltpu.make_async_copy(dst_hbm.at[pl.ds(base, ec)], didx[b], semi[b]).wait()
                    pltpu.async_copy(m_hbm.at[sidx[b]], rows[b], semg[b])

            for b in range(pipe):
                ck = chunk(b)

                @pl.when(ck < nec)
                def _(ck=ck, b=b):
                    pltpu.make_async_copy(m_hbm.at[sidx[b]], rows[b], semg[b]).wait()
                    pltpu.async_copy(rows[b], acc.at[didx[b]], semw[b], add=True)

            for b in range(pipe):
                ck = chunk(b)

                @pl.when(ck < nec)
                def _(ck=ck, b=b):
                    pltpu.make_async_copy(rows[b], acc.at[didx[b]], semw[b]).wait()

            return c

        lax.fori_loop(0, ng, ebody, 0)
        plsc.subcore_barrier()
        _fire_drain_rows(lambda ck: acc.at[pl.ds(ck * rc, rc)],
                         lambda ck: out_hbm.at[pl.ds(cid * N + ck * rc, rc)],
                         semz, nrc, rc, sid)

    return k


_DEG_STREAMS = [(E0, N0, 80), (E0, N0, 80), (E1, N1, 80),
                (E1, N1, 80), (E2, N2P, 64), (E2, N2P, 64)]


@functools.partial(
    pl.kernel,
    mesh=plsc.VectorSubcoreMesh(**_MESH),
    out_type=[jax.ShapeDtypeStruct((2 * n, DW), jnp.float32)
              for (_, n, _) in _DEG_STREAMS],
    scratch_types=(
        [pltpu.VMEM((EC,), jnp.int32) for _ in range(PIPE)]
        + [pltpu.VMEM((EC, DW), jnp.float32), pltpu.VMEM((EC, DW), jnp.float32)]
        + [pltpu.VMEM_SHARED((n, DW), jnp.float32) for (_, n, _) in _DEG_STREAMS]
        + [pltpu.SemaphoreType.DMA for _ in range(2 * PIPE + 1)]
    ),
    compiler_params=pltpu.CompilerParams(use_tc_tiling_on_sc=False),
)
def _sc_deg(i0s, i0d, i1s, i1d, i2s, i2d, ones_hbm, zer_hbm,
            o0s, o0d, o1s, o1d, o2s, o2d, *sc):
    idxb = sc[0:PIPE]
    onesb = sc[PIPE]
    zb = sc[PIPE + 1]
    accs = sc[PIPE + 2:PIPE + 8]
    semi = sc[PIPE + 8:2 * PIPE + 8]
    semw = sc[2 * PIPE + 8:3 * PIPE + 8]
    semz = sc[3 * PIPE + 8]
    cid = lax.axis_index("c")
    sid = lax.axis_index("s")
    wid = sid * 2 + cid
    pltpu.sync_copy(ones_hbm, onesb)
    pltpu.sync_copy(zer_hbm, zb)
    idxs = [i0s, i0d, i1s, i1d, i2s, i2d]
    outs = [o0s, o0d, o1s, o1d, o2s, o2d]
    for (e, n, rc), acc in zip(_DEG_STREAMS, accs):
        _fire_drain_rows(lambda ck, rc=rc: zb.at[pl.ds(0, rc)],
                         lambda ck, acc=acc, rc=rc: acc.at[pl.ds(ck * rc, rc)],
                         semz, n // rc, rc, sid)
    plsc.subcore_barrier()
    for (e, n, rc), idx, acc in zip(_DEG_STREAMS, idxs, accs):
        nec = e // EC
        ng = _cdiv(_cdiv(nec, NW), PIPE)

        def ebody(g, c, idx=idx, acc=acc, nec=nec):
            def chunk(b):
                return (g * PIPE + b) * NW + wid

            for b in range(PIPE):
                ck = chunk(b)

                @pl.when(ck < nec)
                def _(ck=ck, b=b):
                    pltpu.async_copy(idx.at[pl.ds(ck * EC, EC)], idxb[b], semi[b])

            for b in range(PIPE):
                ck = chunk(b)

                @pl.when(ck < nec)
                def _(ck=ck, b=b):
                    pltpu.make_async_copy(idx.at[pl.ds(ck * EC, EC)], idxb[b], semi[b]).wait()
                    pltpu.async_copy(onesb, acc.at[idxb[b]], semw[b], add=True)

            for b in range(PIPE):
                ck = chunk(b)

                @pl.when(ck < nec)
                def _(ck=ck, b=b):
                    pltpu.make_async_copy(onesb, acc.at[idxb[b]], semw[b]).wait()

            return c

        lax.fori_loop(0, ng, ebody, 0)
    plsc.subcore_barrier()
    for (e, n, rc), acc, out in zip(_DEG_STREAMS, accs, outs):
        _fire_drain_rows(lambda ck, acc=acc, rc=rc: acc.at[pl.ds(ck * rc, rc)],
                         lambda ck, out=out, rc=rc, n=n: out.at[pl.ds(cid * n + ck * rc, rc)],
                         semz, n // rc, rc, sid)


def _norm(dref):
    d = dref[0, :, 0:1] + dref[1, :, 0:1]
    return jnp.where(d > 0, lax.rsqrt(jnp.maximum(d, 1.0)), 0.0)


def _tc_scale_m0(features, dS0):
    R = 400

    def body(fref, dref, oref):
        oref[...] = fref[...] * _norm(dref)

    return pl.pallas_call(
        body,
        grid=(N0 // R,),
        in_specs=[pl.BlockSpec((R, D), lambda i: (i, 0)),
                  pl.BlockSpec((2, R, DW), lambda i: (0, i, 0))],
        out_specs=pl.BlockSpec((R, D), lambda i: (i, 0)),
        out_shape=jax.ShapeDtypeStruct((N0, D), jnp.float32),
    )(features, dS0)


def _tc_gcn(aggp, dD, W, b, relu, N, Dout, R):
    def body(aref, dref, wref, bref, oref):
        a = (aref[0] + aref[1]) * _norm(dref)
        o = jnp.dot(a, wref[...], preferred_element_type=jnp.float32) + bref[...]
        if relu:
            o = jnp.maximum(o, 0.0)
        oref[...] = o

    return pl.pallas_call(
        body,
        grid=(N // R,),
        in_specs=[pl.BlockSpec((2, R, D), lambda i: (0, i, 0)),
                  pl.BlockSpec((2, R, DW), lambda i: (0, i, 0)),
                  pl.BlockSpec((D, Dout), lambda i: (0, 0)),
                  pl.BlockSpec((1, Dout), lambda i: (0, 0))],
        out_specs=pl.BlockSpec((R, Dout), lambda i: (i, 0)),
        out_shape=jax.ShapeDtypeStruct((N, Dout), jnp.float32),
    )(aggp, dD, W, b)


def _tc_gcn_proj(aggp, dD, W, b, P, dS, Nout, K, Kb):
    nk = K // Kb

    def body(aref, dref, wref, bref, pref, sref, oref):
        kk = pl.program_id(0)
        a = (aref[0] + aref[1]) * _norm(dref)
        h = jnp.maximum(
            jnp.dot(a, wref[...], preferred_element_type=jnp.float32)
            + bref[...], 0.0)
        t = lax.dot_general(pref[...], h, (((0,), (0,)), ((), ())),
                            preferred_element_type=jnp.float32)

        @pl.when(kk == 0)
        def _():
            oref[...] = t

        @pl.when(kk > 0)
        def _():
            oref[...] += t

        @pl.when(kk == nk - 1)
        def _():
            oref[...] *= _norm(sref)

    return pl.pallas_call(
        body,
        grid=(nk,),
        in_specs=[pl.BlockSpec((2, Kb, D), lambda k: (0, k, 0)),
                  pl.BlockSpec((2, Kb, DW), lambda k: (0, k, 0)),
                  pl.BlockSpec((D, D), lambda k: (0, 0)),
                  pl.BlockSpec((1, D), lambda k: (0, 0)),
                  pl.BlockSpec((Kb, Nout), lambda k: (k, 0)),
                  pl.BlockSpec((2, Nout, DW), lambda k: (0, 0, 0))],
        out_specs=pl.BlockSpec((Nout, D), lambda k: (0, 0)),
        out_shape=jax.ShapeDtypeStruct((Nout, D), jnp.float32),
    )(aggp, dD, W, b, P, dS)


def _tc_gcn_proj_emb(aggp, dD, W, b, P2p, dS2p, W2):
    def body(aref, dref, wref, bref, pref, sref, w2ref, eref, mref):
        a = (aref[0] + aref[1]) * _norm(dref)
        h = jnp.maximum(
            jnp.dot(a, wref[...], preferred_element_type=jnp.float32)
            + bref[...], 0.0)
        e = lax.dot_general(pref[...], h, (((0,), (0,)), ((), ())),
                            preferred_element_type=jnp.float32)
        eref[...] = e
        mref[...] = jnp.dot(e * _norm(sref), w2ref[...],
                            preferred_element_type=jnp.float32)

    return pl.pallas_call(
        body,
        grid=(1,),
        in_specs=[pl.BlockSpec((2, N1, D), lambda i: (0, 0, 0)),
                  pl.BlockSpec((2, N1, DW), lambda i: (0, 0, 0)),
                  pl.BlockSpec((D, D), lambda i: (0, 0)),
                  pl.BlockSpec((1, D), lambda i: (0, 0)),
                  pl.BlockSpec((N1, N2P), lambda i: (0, 0)),
                  pl.BlockSpec((2, N2P, DW), lambda i: (0, 0, 0)),
                  pl.BlockSpec((D, 64), lambda i: (0, 0))],
        out_specs=[pl.BlockSpec((N2P, D), lambda i: (0, 0)),
                   pl.BlockSpec((N2P, 64), lambda i: (0, 0))],
        out_shape=[jax.ShapeDtypeStruct((N2P, D), jnp.float32),
                   jax.ShapeDtypeStruct((N2P, 64), jnp.float32)],
    )(aggp, dD, W, b, P2p, dS2p, W2)


def _tc_final(aggp, dD, b):
    def body(aref, dref, bref, oref):
        oref[...] = (aref[0] + aref[1]) * _norm(dref) + bref[...]

    return pl.pallas_call(
        body,
        grid=(1,),
        in_specs=[pl.BlockSpec((2, N2P, 64), lambda i: (0, 0, 0)),
                  pl.BlockSpec((2, N2P, DW), lambda i: (0, 0, 0)),
                  pl.BlockSpec((1, 64), lambda i: (0, 0))],
        out_specs=pl.BlockSpec((N2P, 64), lambda i: (0, 0)),
        out_shape=jax.ShapeDtypeStruct((N2P, 64), jnp.float32),
    )(aggp, dD, b)


def kernel(features, edge_index0, edge_index1, edge_index2, P1, P2,
           W0, b0, W1, b1, W2, b2):
    f32 = jnp.float32
    src0 = edge_index0[0].astype(jnp.int32)
    dst0 = edge_index0[1].astype(jnp.int32)
    src1 = edge_index1[0].astype(jnp.int32)
    dst1 = edge_index1[1].astype(jnp.int32)
    src2 = edge_index2[0].astype(jnp.int32)
    dst2 = edge_index2[1].astype(jnp.int32)
    z128 = jnp.zeros((EC, D), f32)
    z16 = jnp.zeros((EC, DW), f32)
    o16 = jnp.ones((EC, DW), f32)

    degs = _sc_deg(src0, dst0, src1, dst1, src2, dst2, o16, z16)
    dS0, dD0, dS1, dD1, dS2p, dD2p = [x.reshape(2, -1, DW) for x in degs]

    z64 = jnp.zeros((64, D), f32)
    m0 = _tc_scale_m0(features, dS0)
    z40 = jnp.zeros((40, D), f32)
    agg0 = _sc_agg(N0, E0, 40, 40, 9)(m0, src0, dst0, z40).reshape(2, N0, D)
    m1 = _tc_gcn_proj(agg0, dD0, W0, b0.reshape(1, -1), P1, dS1, N1, N0, 1000)
    agg1 = _sc_agg(N1, E1, 80, EC, 6)(m1, src1, dst1, z128).reshape(2, N1, D)
    P2p = jnp.pad(P2, ((0, 0), (0, N2P - N2)))
    emb_p, m2w = _tc_gcn_proj_emb(agg1, dD1, W1, b1.reshape(1, -1), P2p,
                                  dS2p, W2)
    z64w = jnp.zeros((EC, 64), f32)
    agg2p = _sc_agg(N2P, E2, 64, EC, 6, 64)(m2w, src2, dst2,
                                            z64w).reshape(2, N2P, 64)
    outp = _tc_final(agg2p, dD2p, b2.reshape(1, -1))
    return (outp[:N2], emb_p[:N2])
